# baseline (speedup 1.0000x reference)
import functools

import jax
import jax.numpy as jnp
from jax import lax
from jax.experimental import pallas as pl
from jax.experimental.pallas import tpu as pltpu
from jax.experimental.pallas import tpu_sc as plsc

E = 64
TOP_K = 2
H = 2048
I = 768
T = 2048
NP = T * TOP_K
BLK = 128
MAXB = 96
NROWS = MAXB * BLK
NIT = 3
ITILE = I // NIT

NSC = 32
PAIRS_PER_W = NP // NSC
CHUNK = 16


def _routing_body(sel_ref, pos_ref, counts_ref, sched_ref, cum_ref):
    nchunks = NP // BLK
    rows_i = lax.broadcasted_iota(jnp.int32, (BLK, BLK), 0)
    cols_i = lax.broadcasted_iota(jnp.int32, (BLK, BLK), 1)
    ltri = (rows_i >= cols_i).astype(jnp.float32)
    lane = lax.broadcasted_iota(jnp.int32, (BLK, 128), 1)

    def onehot(c):
        e = sel_ref[pl.ds(c * BLK, BLK), :]
        return (e == lane).astype(jnp.float32)

    def pass1(c, carry):
        oh = onehot(c)
        incl = jnp.dot(ltri, oh, preferred_element_type=jnp.float32) + carry
        cum_ref[pl.ds(c * BLK, BLK), :] = incl
        return incl[BLK - 1:BLK, :]

    counts = lax.fori_loop(0, nchunks, pass1, jnp.zeros((1, 128), jnp.float32))
    counts_ref[...] = jnp.broadcast_to(counts, (8, 128))

    padded = jnp.ceil(counts / float(BLK)) * float(BLK)
    sm_r = lax.broadcasted_iota(jnp.int32, (128, 128), 0)
    sm_c = lax.broadcasted_iota(jnp.int32, (128, 128), 1)
    strict = (sm_r < sm_c).astype(jnp.float32)
    po = jnp.dot(padded, strict, preferred_element_type=jnp.float32)

    def pass2(c, carry):
        oh = onehot(c)
        incl = cum_ref[pl.ds(c * BLK, BLK), :]
        val = oh * (incl + po - 1.0)
        slot = jnp.sum(val, axis=1, keepdims=True)
        pos_ref[pl.ds(c * BLK, BLK), :] = jnp.broadcast_to(
            slot.astype(jnp.int32), (BLK, 128))
        return carry

    lax.fori_loop(0, nchunks, pass2, jnp.int32(0))

    nb = jnp.ceil(counts / float(BLK))
    incl = (sm_c <= sm_r).astype(jnp.float32)
    ones_col = jnp.ones((128, 1), jnp.float32)
    ones_row = jnp.ones((1, 128), jnp.float32)
    nbB = jnp.broadcast_to(nb, (128, 128))
    cb_col = jnp.dot(nbB * incl, ones_col,
                     preferred_element_type=jnp.float32)
    tot = cb_col[127:128, 0:1]
    tot_col = jnp.dot(ones_col, tot,
                      preferred_element_type=jnp.float32)
    tot_row = jnp.dot(tot, ones_row,
                      preferred_element_type=jnp.float32)
    cbB = jnp.broadcast_to(cb_col, (128, 128)).astype(jnp.int32)
    laneBi = lax.broadcasted_iota(jnp.int32, (128, 128), 1)
    cnt = jnp.dot(ones_row, (cbB <= laneBi).astype(jnp.float32),
                  preferred_element_type=jnp.float32)
    le = jnp.dot(ones_row,
                 (cbB < jnp.broadcast_to(tot_col, (128, 128)).astype(
                     jnp.int32)).astype(jnp.float32),
                 preferred_element_type=jnp.float32)
    active = (lax.broadcasted_iota(jnp.int32, (1, 128), 1)
              < tot_row.astype(jnp.int32))
    be = jnp.where(active, cnt, le)
    sched_ref[0:1, :] = be.astype(jnp.int32)
    sched_ref[1:2, :] = active.astype(jnp.int32)


def _routing(sel_flat):
    return pl.pallas_call(
        _routing_body,
        out_shape=(
            jax.ShapeDtypeStruct((NP, 128), jnp.int32),
            jax.ShapeDtypeStruct((8, 128), jnp.float32),
            jax.ShapeDtypeStruct((8, 128), jnp.int32),
        ),
        scratch_shapes=[pltpu.VMEM((NP, 128), jnp.float32)],
    )(sel_flat)


def _gather_group(hidden_f32, tok_idx, pos_idx):
    mesh = plsc.VectorSubcoreMesh(core_axis_name="c", subcore_axis_name="s")

    @functools.partial(
        pl.kernel,
        out_type=jax.ShapeDtypeStruct((NROWS, H), jnp.float32),
        mesh=mesh,
        scratch_types=[
            pltpu.VMEM((PAIRS_PER_W // CHUNK, CHUNK), jnp.int32),
            pltpu.VMEM((PAIRS_PER_W // CHUNK, CHUNK), jnp.int32),
            pltpu.VMEM((CHUNK, H), jnp.float32),
            pltpu.VMEM((CHUNK, H), jnp.float32),
            pltpu.SemaphoreType.DMA,
            pltpu.SemaphoreType.DMA,
        ],
    )
    def k(hid_hbm, tok_hbm, pos_hbm, xs_hbm, tok_v, pos_v, rows_a, rows_b, sg, ss):
        c = lax.axis_index("c")
        s = lax.axis_index("s")
        w = s * 2 + c
        pltpu.sync_copy(tok_hbm.at[w], tok_v)
        pltpu.sync_copy(pos_hbm.at[w], pos_v)
        nch = PAIRS_PER_W // CHUNK
        bufs = (rows_a, rows_b)
        gathers = [None] * nch
        scatters = [None] * nch
        gathers[0] = pltpu.async_copy(hid_hbm.at[tok_v.at[0]], bufs[0], sg)
        for ch in range(nch):
            gathers[ch].wait()
            scatters[ch] = pltpu.async_copy(
                bufs[ch % 2], xs_hbm.at[pos_v.at[ch]], ss)
            if ch + 1 < nch:
                if ch >= 1:
                    scatters[ch - 1].wait()
                gathers[ch + 1] = pltpu.async_copy(
                    hid_hbm.at[tok_v.at[ch + 1]], bufs[(ch + 1) % 2], sg)
        scatters[nch - 2].wait()
        scatters[nch - 1].wait()

    return k(hidden_f32, tok_idx, pos_idx)


def _ffn_body(expert_sref, active_sref, xs_ref, g_ref, u_ref, d_ref, ys_ref):
    b = pl.program_id(0)

    @pl.when(active_sref[b] > 0)
    def _():
        x = xs_ref[...].astype(jnp.bfloat16)
        g = lax.dot_general(x, g_ref[0].astype(jnp.bfloat16),
                            (((1,), (1,)), ((), ())),
                            preferred_element_type=jnp.float32)
        u = lax.dot_general(x, u_ref[0].astype(jnp.bfloat16),
                            (((1,), (1,)), ((), ())),
                            preferred_element_type=jnp.float32)
        h = (g * jax.nn.sigmoid(g) * u).astype(jnp.bfloat16)
        ys_ref[...] = lax.dot_general(h, d_ref[0].astype(jnp.bfloat16),
                                      (((1,), (1,)), ((), ())),
                                      preferred_element_type=jnp.float32)


def _ffn(xs, gate_w, up_w, down_w, block_expert, block_active):
    grid_spec = pltpu.PrefetchScalarGridSpec(
        num_scalar_prefetch=2,
        grid=(MAXB,),
        in_specs=[
            pl.BlockSpec((BLK, H), lambda b, es, as_: (b * as_[b], 0)),
            pl.BlockSpec((1, I, H), lambda b, es, as_: (es[b], 0, 0)),
            pl.BlockSpec((1, I, H), lambda b, es, as_: (es[b], 0, 0)),
            pl.BlockSpec((1, H, I), lambda b, es, as_: (es[b], 0, 0)),
        ],
        out_specs=pl.BlockSpec(
            (BLK, H),
            lambda b, es, as_: (jnp.where(as_[b] > 0, b, MAXB), 0)),
    )
    return pl.pallas_call(
        _ffn_body,
        grid_spec=grid_spec,
        out_shape=jax.ShapeDtypeStruct((NROWS + BLK, H), jnp.float32),
    )(block_expert, block_active, xs, gate_w, up_w, down_w)


def _combine(ys, pm_idx, rw_grp):
    mesh = plsc.VectorSubcoreMesh(core_axis_name="c", subcore_axis_name="s")
    tok_per_w = T // NSC
    tok_per_ch = CHUNK // TOP_K
    nch = tok_per_w // tok_per_ch

    @functools.partial(
        pl.kernel,
        out_type=jax.ShapeDtypeStruct((T, H), jnp.float32),
        mesh=mesh,
        scratch_types=[
            pltpu.VMEM((nch, CHUNK), jnp.int32),
            pltpu.VMEM((nch, CHUNK), jnp.float32),
            pltpu.VMEM((CHUNK, H), jnp.float32),
            pltpu.VMEM((CHUNK, H), jnp.float32),
            pltpu.VMEM((tok_per_ch, H), jnp.float32),
            pltpu.SemaphoreType.DMA,
        ],
    )
    def k(ys_hbm, pm_hbm, rw_hbm, out_hbm, pm_v, rw_v, rows_a, rows_b, acc_v, sg):
        c = lax.axis_index("c")
        s = lax.axis_index("s")
        w = s * 2 + c
        pltpu.sync_copy(pm_hbm.at[w], pm_v)
        pltpu.sync_copy(rw_hbm.at[w], rw_v)
        bufs = (rows_a, rows_b)
        gathers = [None] * nch
        gathers[0] = pltpu.async_copy(ys_hbm.at[pm_v.at[0]], bufs[0], sg)
        for ch in range(nch):
            gathers[ch].wait()
            if ch + 1 < nch:
                gathers[ch + 1] = pltpu.async_copy(
                    ys_hbm.at[pm_v.at[ch + 1]], bufs[(ch + 1) % 2], sg)
            rows_v = bufs[ch % 2]
            wrow = rw_v[ch, :]
            w0 = [wrow[2 * i] for i in range(tok_per_ch)]
            w1 = [wrow[2 * i + 1] for i in range(tok_per_ch)]

            def inner(j, carry):
                sl = pl.ds(j * 16, 16)
                for i in range(tok_per_ch):
                    acc_v[i, sl] = rows_v[2 * i, sl] * w0[i] + \
                        rows_v[2 * i + 1, sl] * w1[i]
                return carry

            lax.fori_loop(0, H // 16, inner, jnp.int32(0))
            pltpu.sync_copy(
                acc_v, out_hbm.at[pl.ds(w * tok_per_w + ch * tok_per_ch,
                                        tok_per_ch)])

    return k(ys, pm_idx, rw_grp)


def kernel(hidden_states, gate_proj, up_proj, down_proj, routing_weights,
           selected_experts):
    sel = selected_experts.astype(jnp.int32)
    sel_flat = jnp.concatenate([sel[:, 0], sel[:, 1]]).reshape(NP, 1)

    pos_full, counts_full, sched = _routing(sel_flat)
    pos = pos_full[:, 0]
    block_expert = sched[0, :MAXB]
    block_active = sched[1, :MAXB]

    tok_idx = (jnp.arange(NP, dtype=jnp.int32) % T).reshape(NSC, PAIRS_PER_W // CHUNK, CHUNK)
    pos_idx = pos.reshape(NSC, PAIRS_PER_W // CHUNK, CHUNK)
    pm = jnp.stack([pos[:T], pos[T:]], axis=1).reshape(NSC, (T // NSC) // (CHUNK // TOP_K), CHUNK)
    rw_grp = routing_weights.astype(jnp.float32).reshape(
        NSC, (T // NSC) // (CHUNK // TOP_K), CHUNK)

    xs = _gather_group(hidden_states, tok_idx, pos_idx)
    ys = _ffn(xs, gate_proj, up_proj, down_proj, block_expert, block_active)
    out = _combine(ys, pm, rw_grp)
    return out


# speedup vs baseline: 1.0088x; 1.0088x over previous
import functools

import jax
import jax.numpy as jnp
from jax import lax
from jax.experimental import pallas as pl
from jax.experimental.pallas import tpu as pltpu
from jax.experimental.pallas import tpu_sc as plsc

E = 64
TOP_K = 2
H = 2048
I = 768
T = 2048
NP = T * TOP_K
BLK = 128
MAXB = 96
NROWS = MAXB * BLK
NIT = 3
ITILE = I // NIT

NSC = 32
PAIRS_PER_W = NP // NSC
CHUNK = 16


def _routing_body(sel_ref, pos_ref, counts_ref, sched_ref, cum_ref):
    nchunks = NP // BLK
    rows_i = lax.broadcasted_iota(jnp.int32, (BLK, BLK), 0)
    cols_i = lax.broadcasted_iota(jnp.int32, (BLK, BLK), 1)
    ltri = (rows_i >= cols_i).astype(jnp.float32)
    lane = lax.broadcasted_iota(jnp.int32, (BLK, 128), 1)

    def onehot(c):
        e = sel_ref[pl.ds(c * BLK, BLK), :]
        return (e == lane).astype(jnp.float32)

    def pass1(c, carry):
        oh = onehot(c)
        incl = jnp.dot(ltri, oh, preferred_element_type=jnp.float32) + carry
        cum_ref[pl.ds(c * BLK, BLK), :] = incl
        return incl[BLK - 1:BLK, :]

    counts = lax.fori_loop(0, nchunks, pass1, jnp.zeros((1, 128), jnp.float32))
    counts_ref[...] = jnp.broadcast_to(counts, (8, 128))

    padded = jnp.ceil(counts / float(BLK)) * float(BLK)
    sm_r = lax.broadcasted_iota(jnp.int32, (128, 128), 0)
    sm_c = lax.broadcasted_iota(jnp.int32, (128, 128), 1)
    strict = (sm_r < sm_c).astype(jnp.float32)
    po = jnp.dot(padded, strict, preferred_element_type=jnp.float32)

    def pass2(c, carry):
        oh = onehot(c)
        incl = cum_ref[pl.ds(c * BLK, BLK), :]
        val = oh * (incl + po - 1.0)
        slot = jnp.sum(val, axis=1, keepdims=True)
        pos_ref[pl.ds(c * BLK, BLK), :] = jnp.broadcast_to(
            slot.astype(jnp.int32), (BLK, 128))
        return carry

    lax.fori_loop(0, nchunks, pass2, jnp.int32(0))

    nb = jnp.ceil(counts / float(BLK))
    incl = (sm_c <= sm_r).astype(jnp.float32)
    ones_col = jnp.ones((128, 1), jnp.float32)
    ones_row = jnp.ones((1, 128), jnp.float32)
    nbB = jnp.broadcast_to(nb, (128, 128))
    cb_col = jnp.dot(nbB * incl, ones_col,
                     preferred_element_type=jnp.float32)
    tot = cb_col[127:128, 0:1]
    tot_col = jnp.dot(ones_col, tot,
                      preferred_element_type=jnp.float32)
    tot_row = jnp.dot(tot, ones_row,
                      preferred_element_type=jnp.float32)
    cbB = jnp.broadcast_to(cb_col, (128, 128)).astype(jnp.int32)
    laneBi = lax.broadcasted_iota(jnp.int32, (128, 128), 1)
    cnt = jnp.dot(ones_row, (cbB <= laneBi).astype(jnp.float32),
                  preferred_element_type=jnp.float32)
    le = jnp.dot(ones_row,
                 (cbB < jnp.broadcast_to(tot_col, (128, 128)).astype(
                     jnp.int32)).astype(jnp.float32),
                 preferred_element_type=jnp.float32)
    active = (lax.broadcasted_iota(jnp.int32, (1, 128), 1)
              < tot_row.astype(jnp.int32))
    be = jnp.where(active, cnt, le)
    sched_ref[0:1, :] = be.astype(jnp.int32)
    sched_ref[1:2, :] = active.astype(jnp.int32)


def _routing(sel_flat):
    return pl.pallas_call(
        _routing_body,
        out_shape=(
            jax.ShapeDtypeStruct((NP, 128), jnp.int32),
            jax.ShapeDtypeStruct((8, 128), jnp.float32),
            jax.ShapeDtypeStruct((8, 128), jnp.int32),
        ),
        scratch_shapes=[pltpu.VMEM((NP, 128), jnp.float32)],
    )(sel_flat)


def _gather_group(hidden_f32, pos_idx):
    mesh = plsc.VectorSubcoreMesh(core_axis_name="c", subcore_axis_name="s")

    @functools.partial(
        pl.kernel,
        out_type=jax.ShapeDtypeStruct((NROWS, H), jnp.float32),
        mesh=mesh,
        scratch_types=[
            pltpu.VMEM((PAIRS_PER_W // CHUNK, CHUNK), jnp.int32),
            pltpu.VMEM((CHUNK, H), jnp.float32),
            pltpu.VMEM((CHUNK, H), jnp.float32),
            pltpu.VMEM((CHUNK, H), jnp.float32),
            pltpu.SemaphoreType.DMA,
            pltpu.SemaphoreType.DMA,
        ],
    )
    def k(hid_hbm, pos_hbm, xs_hbm, pos_v, rows_a, rows_b, rows_c, sg, ss):
        c = lax.axis_index("c")
        s = lax.axis_index("s")
        w = s * 2 + c
        pltpu.sync_copy(pos_hbm.at[w], pos_v)
        nch = PAIRS_PER_W // CHUNK
        bufs = (rows_a, rows_b, rows_c)
        base = (w % (T // PAIRS_PER_W)) * PAIRS_PER_W

        def rd(ch):
            return pltpu.async_copy(
                hid_hbm.at[pl.ds(base + ch * CHUNK, CHUNK)],
                bufs[ch % 3], sg)

        gathers = [None] * nch
        scatters = [None] * nch
        gathers[0] = rd(0)
        gathers[1] = rd(1)
        for ch in range(nch):
            gathers[ch].wait()
            scatters[ch] = pltpu.async_copy(
                bufs[ch % 3], xs_hbm.at[pos_v.at[ch]], ss)
            if ch + 2 < nch:
                if ch >= 1:
                    scatters[ch - 1].wait()
                gathers[ch + 2] = rd(ch + 2)
        for ch in range(nch - 3, nch):
            scatters[ch].wait()

    return k(hidden_f32, pos_idx)


def _ffn_body(expert_sref, active_sref, xs_ref, g_ref, u_ref, d_ref, ys_ref):
    b = pl.program_id(0)

    @pl.when(active_sref[b] > 0)
    def _():
        x = xs_ref[...].astype(jnp.bfloat16)
        g = lax.dot_general(x, g_ref[0].astype(jnp.bfloat16),
                            (((1,), (1,)), ((), ())),
                            preferred_element_type=jnp.float32)
        u = lax.dot_general(x, u_ref[0].astype(jnp.bfloat16),
                            (((1,), (1,)), ((), ())),
                            preferred_element_type=jnp.float32)
        h = (g * jax.nn.sigmoid(g) * u).astype(jnp.bfloat16)
        ys_ref[...] = lax.dot_general(h, d_ref[0].astype(jnp.bfloat16),
                                      (((1,), (1,)), ((), ())),
                                      preferred_element_type=jnp.float32)


def _ffn(xs, gate_w, up_w, down_w, block_expert, block_active):
    grid_spec = pltpu.PrefetchScalarGridSpec(
        num_scalar_prefetch=2,
        grid=(MAXB,),
        in_specs=[
            pl.BlockSpec((BLK, H), lambda b, es, as_: (b * as_[b], 0)),
            pl.BlockSpec((1, I, H), lambda b, es, as_: (es[b], 0, 0)),
            pl.BlockSpec((1, I, H), lambda b, es, as_: (es[b], 0, 0)),
            pl.BlockSpec((1, H, I), lambda b, es, as_: (es[b], 0, 0)),
        ],
        out_specs=pl.BlockSpec(
            (BLK, H),
            lambda b, es, as_: (jnp.where(as_[b] > 0, b, MAXB), 0)),
    )
    return pl.pallas_call(
        _ffn_body,
        grid_spec=grid_spec,
        out_shape=jax.ShapeDtypeStruct((NROWS + BLK, H), jnp.float32),
    )(block_expert, block_active, xs, gate_w, up_w, down_w)


def _combine(ys, pm_idx, rw_grp):
    mesh = plsc.VectorSubcoreMesh(core_axis_name="c", subcore_axis_name="s")
    tok_per_w = T // NSC
    tok_per_ch = CHUNK // TOP_K
    nch = tok_per_w // tok_per_ch

    @functools.partial(
        pl.kernel,
        out_type=jax.ShapeDtypeStruct((T, H), jnp.float32),
        mesh=mesh,
        scratch_types=[
            pltpu.VMEM((nch, CHUNK), jnp.int32),
            pltpu.VMEM((nch, CHUNK), jnp.float32),
            pltpu.VMEM((CHUNK, H), jnp.float32),
            pltpu.VMEM((CHUNK, H), jnp.float32),
            pltpu.VMEM((CHUNK, H), jnp.float32),
            pltpu.VMEM((tok_per_ch, H), jnp.float32),
            pltpu.SemaphoreType.DMA,
        ],
    )
    def k(ys_hbm, pm_hbm, rw_hbm, out_hbm, pm_v, rw_v, rows_a, rows_b, rows_c,
          acc_v, sg):
        c = lax.axis_index("c")
        s = lax.axis_index("s")
        w = s * 2 + c
        pltpu.sync_copy(pm_hbm.at[w], pm_v)
        pltpu.sync_copy(rw_hbm.at[w], rw_v)
        bufs = (rows_a, rows_b, rows_c)
        gathers = [None] * nch
        gathers[0] = pltpu.async_copy(ys_hbm.at[pm_v.at[0]], bufs[0], sg)
        gathers[1] = pltpu.async_copy(ys_hbm.at[pm_v.at[1]], bufs[1], sg)
        for ch in range(nch):
            gathers[ch].wait()
            if ch + 2 < nch:
                gathers[ch + 2] = pltpu.async_copy(
                    ys_hbm.at[pm_v.at[ch + 2]], bufs[(ch + 2) % 3], sg)
            rows_v = bufs[ch % 3]
            wrow = rw_v[ch, :]
            w0 = [wrow[2 * i] for i in range(tok_per_ch)]
            w1 = [wrow[2 * i + 1] for i in range(tok_per_ch)]

            def inner(j, carry):
                sl = pl.ds(j * 16, 16)
                for i in range(tok_per_ch):
                    acc_v[i, sl] = rows_v[2 * i, sl] * w0[i] + \
                        rows_v[2 * i + 1, sl] * w1[i]
                return carry

            lax.fori_loop(0, H // 16, inner, jnp.int32(0))
            pltpu.sync_copy(
                acc_v, out_hbm.at[pl.ds(w * tok_per_w + ch * tok_per_ch,
                                        tok_per_ch)])

    return k(ys, pm_idx, rw_grp)


def kernel(hidden_states, gate_proj, up_proj, down_proj, routing_weights,
           selected_experts):
    sel = selected_experts.astype(jnp.int32)
    sel_flat = jnp.concatenate([sel[:, 0], sel[:, 1]]).reshape(NP, 1)

    pos_full, counts_full, sched = _routing(sel_flat)
    pos = pos_full[:, 0]
    block_expert = sched[0, :MAXB]
    block_active = sched[1, :MAXB]

    pos_idx = pos.reshape(NSC, PAIRS_PER_W // CHUNK, CHUNK)
    pm = jnp.stack([pos[:T], pos[T:]], axis=1).reshape(NSC, (T // NSC) // (CHUNK // TOP_K), CHUNK)
    rw_grp = routing_weights.astype(jnp.float32).reshape(
        NSC, (T // NSC) // (CHUNK // TOP_K), CHUNK)

    xs = _gather_group(hidden_states, pos_idx)
    ys = _ffn(xs, gate_proj, up_proj, down_proj, block_expert, block_active)
    out = _combine(ys, pm, rw_grp)
    return out
